# baseline (speedup 1.0000x reference)
import functools

import jax
import jax.numpy as jnp
from jax import lax
from jax.experimental import pallas as pl
from jax.experimental.pallas import tpu as pltpu
from jax.experimental.pallas import tpu_sc as plsc

_HID = 16
_NW = 32
_CH = 128
_LANES = 16


def _ceil_to(v, m):
    return (v + m - 1) // m * m


def _sc_mesh():
    return plsc.VectorSubcoreMesh(core_axis_name="c", subcore_axis_name="s")


_SC_PARAMS = pltpu.CompilerParams(use_tc_tiling_on_sc=False)


def _sc_gather(table, idx2d):
    nch_total, _ = idx2d.shape
    w = table.shape[1]
    ep = nch_total * _CH
    per = ep // _NW
    nch = per // _CH

    @functools.partial(
        pl.kernel,
        out_type=jax.ShapeDtypeStruct((ep, w), jnp.float32),
        mesh=_sc_mesh(),
        scratch_types=[
            pltpu.VMEM((nch, _CH), jnp.int32),
            pltpu.VMEM((per, w), jnp.float32),
            pltpu.SemaphoreType.DMA,
            pltpu.SemaphoreType.DMA,
        ],
        compiler_params=_SC_PARAMS,
    )
    def k(table_hbm, idx_hbm, out_hbm, idx_v, rows_v, sem, sem_st):
        wid = lax.axis_index("s") * 2 + lax.axis_index("c")
        base = wid * per
        hn = nch // 2
        hrows = hn * _CH
        pltpu.sync_copy(idx_hbm.at[pl.ds(wid * nch, nch)], idx_v)

        def issue(j, carry):
            pltpu.async_copy(table_hbm.at[idx_v.at[j]],
                             rows_v.at[pl.ds(j * _CH, _CH)], sem)
            return carry

        lax.fori_loop(0, hn, issue, 0)
        pltpu.make_async_copy(table_hbm.at[pl.ds(0, hrows)],
                              rows_v.at[pl.ds(0, hrows)], sem).wait()
        pltpu.async_copy(rows_v.at[pl.ds(0, hrows)],
                         out_hbm.at[pl.ds(base, hrows)], sem_st)
        lax.fori_loop(hn, nch, issue, 0)
        pltpu.make_async_copy(table_hbm.at[pl.ds(0, hrows)],
                              rows_v.at[pl.ds(0, hrows)], sem).wait()
        pltpu.async_copy(rows_v.at[pl.ds(hrows, hrows)],
                         out_hbm.at[pl.ds(base + hrows, hrows)], sem_st)
        pltpu.make_async_copy(rows_v.at[pl.ds(0, hrows)],
                              out_hbm.at[pl.ds(0, hrows)], sem_st).wait()
        pltpu.make_async_copy(rows_v.at[pl.ds(0, hrows)],
                              out_hbm.at[pl.ds(0, hrows)], sem_st).wait()

    return k(table, idx2d)


def _sc_scatter(msg, dst, npad, wide):
    nch_total = dst.shape[0]
    ep = nch_total * _CH
    per = ep // _NW
    nch = per // _CH
    rpt = npad // 16

    if wide:
        agg_shape = (2 * npad, _HID)
        sh_agg_t = pltpu.VMEM_SHARED((npad, _HID), jnp.float32)
        rows_t = pltpu.VMEM((per, _HID), jnp.float32)
    else:
        agg_shape = (2 * npad,)
        sh_agg_t = pltpu.VMEM_SHARED((npad,), jnp.float32)
        rows_t = pltpu.VMEM((per,), jnp.float32)

    @functools.partial(
        pl.kernel,
        out_type=(jax.ShapeDtypeStruct(agg_shape, jnp.float32),
                  jax.ShapeDtypeStruct((2 * npad,), jnp.float32)),
        mesh=_sc_mesh(),
        scratch_types=[
            pltpu.VMEM((nch, _CH), jnp.int32),
            rows_t,
            pltpu.VMEM((_CH,), jnp.float32),
            pltpu.VMEM((per,), jnp.float32),
            sh_agg_t,
            pltpu.VMEM_SHARED((npad,), jnp.float32),
            pltpu.SemaphoreType.DMA,
            pltpu.SemaphoreType.DMA,
        ],
        compiler_params=_SC_PARAMS,
    )
    def k(msg_hbm, dst_hbm, zrow_hbm, zcnt_hbm, agg_hbm, cnt_hbm,
          idx_v, rows_v, ones_v, dummy_v, sh_agg, sh_cnt, sem_a, sem_c):
        cid = lax.axis_index("c")
        sid = lax.axis_index("s")
        wid = sid * 2 + cid
        base = wid * per

        for i in range(_CH // _LANES):
            ones_v[pl.ds(i * _LANES, _LANES)] = jnp.ones((_LANES,), jnp.float32)

        r0 = sid * rpt
        pltpu.sync_copy(dst_hbm.at[pl.ds(wid * nch, nch)], idx_v)
        msg_cp = pltpu.async_copy(msg_hbm.at[pl.ds(base, per)], rows_v, sem_a)
        if wide:
            pltpu.sync_copy(zrow_hbm.at[pl.ds(r0, rpt)], sh_agg.at[pl.ds(r0, rpt)])
        else:
            pltpu.sync_copy(zcnt_hbm.at[pl.ds(r0, rpt)], sh_agg.at[pl.ds(r0, rpt)])
        pltpu.sync_copy(zcnt_hbm.at[pl.ds(r0, rpt)], sh_cnt.at[pl.ds(r0, rpt)])
        plsc.subcore_barrier()
        msg_cp.wait()

        def body(j, carry):
            sl = pl.ds(j * _CH, _CH)
            pltpu.async_copy(rows_v.at[sl], sh_agg.at[idx_v.at[j]], sem_a,
                             add=True)
            pltpu.async_copy(ones_v, sh_cnt.at[idx_v.at[j]], sem_c, add=True)
            return carry

        lax.fori_loop(0, nch, body, 0)
        pltpu.make_async_copy(msg_hbm.at[pl.ds(base, per)], rows_v,
                              sem_a).wait()
        pltpu.make_async_copy(zcnt_hbm.at[pl.ds(0, per)], dummy_v,
                              sem_c).wait()
        plsc.subcore_barrier()

        o0 = cid * npad + r0
        pltpu.sync_copy(sh_agg.at[pl.ds(r0, rpt)], agg_hbm.at[pl.ds(o0, rpt)])
        pltpu.sync_copy(sh_cnt.at[pl.ds(r0, rpt)], cnt_hbm.at[pl.ds(o0, rpt)])

    zrow = jnp.zeros((npad, _HID), jnp.float32)
    zcnt = jnp.zeros((npad,), jnp.float32)
    return k(msg, dst, zrow, zcnt)


def _tc_msg(stim2d, xs, w1, b1, w2, b2, out_f):
    ep, d = xs.shape[0], w2.shape[0]
    wx = xs.shape[1]
    blk = 2048
    grid = ep // blk

    def body(stim_ref, xs_ref, w1_ref, b1_ref, w2_ref, b2_ref, out_ref):
        s = stim_ref[...]
        h = jnp.maximum(s * w1_ref[...] + b1_ref[...], 0.0)
        w = jnp.dot(h.astype(jnp.bfloat16), w2_ref[...].astype(jnp.bfloat16),
                    preferred_element_type=jnp.float32) + b2_ref[...]
        jj = lax.broadcasted_iota(jnp.int32, (wx, d), 1)
        ii = lax.broadcasted_iota(jnp.int32, (wx, d), 0)
        r = (jj // out_f == ii).astype(jnp.float32)
        xr = jnp.dot(xs_ref[...], r, preferred_element_type=jnp.float32)
        jo = lax.broadcasted_iota(jnp.int32, (d, out_f), 0)
        oo = lax.broadcasted_iota(jnp.int32, (d, out_f), 1)
        sred = (jo % out_f == oo).astype(jnp.float32)
        out_ref[...] = jnp.dot(xr * w, sred,
                               preferred_element_type=jnp.float32)

    return pl.pallas_call(
        body,
        grid=(grid,),
        in_specs=[
            pl.BlockSpec((blk, 1), lambda i: (i, 0)),
            pl.BlockSpec((blk, wx), lambda i: (i, 0)),
            pl.BlockSpec((1, d), lambda i: (0, 0)),
            pl.BlockSpec((1, d), lambda i: (0, 0)),
            pl.BlockSpec((d, d), lambda i: (0, 0)),
            pl.BlockSpec((1, d), lambda i: (0, 0)),
        ],
        out_specs=pl.BlockSpec((blk, out_f), lambda i: (i, 0)),
        out_shape=jax.ShapeDtypeStruct((ep, out_f), jnp.float32),
    )(stim2d, xs, w1, b1.reshape(1, d), w2, b2.reshape(1, d))


def _tc_norm(a0, a1, c0, c1, bias, g, beta):
    npad = a0.shape[0]
    blk = 512

    def body(a0_ref, a1_ref, c0_ref, c1_ref, bi_ref, g_ref, be_ref, out_ref):
        agg = a0_ref[...] + a1_ref[...]
        cnt = jnp.maximum(c0_ref[...] + c1_ref[...], 1.0)
        h = agg / cnt + bi_ref[...]
        mu = jnp.mean(h, axis=-1, keepdims=True)
        var = jnp.mean((h - mu) ** 2, axis=-1, keepdims=True)
        hn = (h - mu) / jnp.sqrt(var + 1e-5) * g_ref[...] + be_ref[...]
        out_ref[...] = jnp.maximum(hn, 0.0)

    return pl.pallas_call(
        body,
        grid=(npad // blk,),
        in_specs=[
            pl.BlockSpec((blk, _HID), lambda i: (i, 0)),
            pl.BlockSpec((blk, _HID), lambda i: (i, 0)),
            pl.BlockSpec((blk, 1), lambda i: (i, 0)),
            pl.BlockSpec((blk, 1), lambda i: (i, 0)),
            pl.BlockSpec((1, _HID), lambda i: (0, 0)),
            pl.BlockSpec((1, _HID), lambda i: (0, 0)),
            pl.BlockSpec((1, _HID), lambda i: (0, 0)),
        ],
        out_specs=pl.BlockSpec((blk, _HID), lambda i: (i, 0)),
        out_shape=jax.ShapeDtypeStruct((npad, _HID), jnp.float32),
    )(a0, a1, c0, c1, bias.reshape(1, _HID), g.reshape(1, _HID),
      beta.reshape(1, _HID))


def _tc_final(a0, a1, c0, c1, bias):
    npad = a0.shape[0]
    blk = 512

    def body(a0_ref, a1_ref, c0_ref, c1_ref, bi_ref, out_ref):
        agg = a0_ref[...] + a1_ref[...]
        cnt = jnp.maximum(c0_ref[...] + c1_ref[...], 1.0)
        h = agg / cnt + bi_ref[...]
        out_ref[...] = jnp.log1p(jnp.exp(-jnp.abs(h))) + jnp.maximum(h, 0.0)

    return pl.pallas_call(
        body,
        grid=(npad // blk,),
        in_specs=[
            pl.BlockSpec((blk, 1), lambda i: (i, 0)),
            pl.BlockSpec((blk, 1), lambda i: (i, 0)),
            pl.BlockSpec((blk, 1), lambda i: (i, 0)),
            pl.BlockSpec((blk, 1), lambda i: (i, 0)),
            pl.BlockSpec((1, 1), lambda i: (0, 0)),
        ],
        out_specs=pl.BlockSpec((blk, 1), lambda i: (i, 0)),
        out_shape=jax.ShapeDtypeStruct((npad, 1), jnp.float32),
    )(a0, a1, c0, c1, bias.reshape(1, 1))


def kernel(x, edge_index1, stim1, edge_index2, stim2, edge_index3, stim3,
           mlp1_W1, mlp1_b1, mlp1_W2, mlp1_b2, conv1_b, norm1_g, norm1_b,
           mlp2_W1, mlp2_b1, mlp2_W2, mlp2_b2, conv2_b, norm2_g, norm2_b,
           mlp3_W1, mlp3_b1, mlp3_W2, mlp3_b2, conv3_b):
    n, in_f = x.shape
    e = edge_index1.shape[1]
    npad = _ceil_to(n + 1, 512)
    epad = _ceil_to(e, _NW * _CH)

    def pad_edges(ei, stim):
        pe = epad - e
        src = jnp.concatenate([ei[0], jnp.zeros((pe,), jnp.int32)])
        dst = jnp.concatenate([ei[1], jnp.full((pe,), n, jnp.int32)])
        st = jnp.concatenate([stim, jnp.zeros((pe,), jnp.float32)])
        return (src.reshape(epad // _CH, _CH), dst.reshape(epad // _CH, _CH),
                st.reshape(epad, 1))

    src1, dst1, st1 = pad_edges(edge_index1, stim1)
    src2, dst2, st2 = pad_edges(edge_index2, stim2)
    src3, dst3, st3 = pad_edges(edge_index3, stim3)

    xpad = jnp.zeros((npad, 8), jnp.float32).at[:n, :in_f].set(x)

    def layer(table, src, dst, st, w1, b1, w2, b2):
        xs = _sc_gather(table, src)
        msg = _tc_msg(st, xs, w1, b1, w2, b2, _HID)
        aggp, cntp = _sc_scatter(msg, dst, npad, wide=True)
        return aggp, cntp

    def split(aggp, cntp, wide):
        if wide:
            a0, a1 = aggp[:npad], aggp[npad:]
        else:
            a0, a1 = aggp[:npad, None], aggp[npad:, None]
        c0, c1 = cntp[:npad, None], cntp[npad:, None]
        return a0, a1, c0, c1

    aggp, cntp = layer(xpad, src1, dst1, st1, mlp1_W1, mlp1_b1, mlp1_W2, mlp1_b2)
    h1 = _tc_norm(*split(aggp, cntp, True), conv1_b, norm1_g, norm1_b)

    aggp, cntp = layer(h1, src2, dst2, st2, mlp2_W1, mlp2_b1, mlp2_W2, mlp2_b2)
    h2 = _tc_norm(*split(aggp, cntp, True), conv2_b, norm2_g, norm2_b)

    xs3 = _sc_gather(h2, src3)
    msg3 = _tc_msg(st3, xs3, mlp3_W1, mlp3_b1, mlp3_W2, mlp3_b2, 1)
    aggp3, cntp3 = _sc_scatter(msg3.reshape(epad), dst3, npad, wide=False)
    out = _tc_final(*split(aggp3, cntp3, False), conv3_b)
    return out[:n]


# speedup vs baseline: 1.0036x; 1.0036x over previous
import functools

import jax
import jax.numpy as jnp
from jax import lax
from jax.experimental import pallas as pl
from jax.experimental.pallas import tpu as pltpu
from jax.experimental.pallas import tpu_sc as plsc

_HID = 16
_NW = 32
_CH = 128
_LANES = 16


def _ceil_to(v, m):
    return (v + m - 1) // m * m


def _sc_mesh():
    return plsc.VectorSubcoreMesh(core_axis_name="c", subcore_axis_name="s")


_SC_PARAMS = pltpu.CompilerParams(use_tc_tiling_on_sc=False)


def _sc_gather(table, idx2d):
    nch_total, _ = idx2d.shape
    w = table.shape[1]
    ep = nch_total * _CH
    per = ep // _NW
    nch = per // _CH

    @functools.partial(
        pl.kernel,
        out_type=jax.ShapeDtypeStruct((ep, w), jnp.float32),
        mesh=_sc_mesh(),
        scratch_types=[
            pltpu.VMEM((nch, _CH), jnp.int32),
            pltpu.VMEM((per, w), jnp.float32),
            pltpu.SemaphoreType.DMA,
            pltpu.SemaphoreType.DMA,
        ],
        compiler_params=_SC_PARAMS,
    )
    def k(table_hbm, idx_hbm, out_hbm, idx_v, rows_v, sem, sem_st):
        wid = lax.axis_index("s") * 2 + lax.axis_index("c")
        base = wid * per
        hn = nch // 2
        hrows = hn * _CH
        pltpu.sync_copy(idx_hbm.at[pl.ds(wid * nch, nch)], idx_v)

        def issue(j, carry):
            pltpu.async_copy(table_hbm.at[idx_v.at[j]],
                             rows_v.at[pl.ds(j * _CH, _CH)], sem)
            return carry

        lax.fori_loop(0, hn, issue, 0)
        pltpu.make_async_copy(table_hbm.at[pl.ds(0, hrows)],
                              rows_v.at[pl.ds(0, hrows)], sem).wait()
        pltpu.async_copy(rows_v.at[pl.ds(0, hrows)],
                         out_hbm.at[pl.ds(base, hrows)], sem_st)
        lax.fori_loop(hn, nch, issue, 0)
        pltpu.make_async_copy(table_hbm.at[pl.ds(0, hrows)],
                              rows_v.at[pl.ds(0, hrows)], sem).wait()
        pltpu.async_copy(rows_v.at[pl.ds(hrows, hrows)],
                         out_hbm.at[pl.ds(base + hrows, hrows)], sem_st)
        pltpu.make_async_copy(rows_v.at[pl.ds(0, hrows)],
                              out_hbm.at[pl.ds(0, hrows)], sem_st).wait()
        pltpu.make_async_copy(rows_v.at[pl.ds(0, hrows)],
                              out_hbm.at[pl.ds(0, hrows)], sem_st).wait()

    return k(table, idx2d)


def _sc_scatter(msg, dst, npad, wide):
    nch_total = dst.shape[0]
    ep = nch_total * _CH
    per = ep // _NW
    nch = per // _CH
    rpt = npad // 16

    if wide:
        agg_shape = (2 * npad, _HID)
        sh_agg_t = pltpu.VMEM_SHARED((npad, _HID), jnp.float32)
        rows_t = pltpu.VMEM((per, _HID), jnp.float32)
    else:
        agg_shape = (2 * npad,)
        sh_agg_t = pltpu.VMEM_SHARED((npad,), jnp.float32)
        rows_t = pltpu.VMEM((per,), jnp.float32)

    @functools.partial(
        pl.kernel,
        out_type=(jax.ShapeDtypeStruct(agg_shape, jnp.float32),
                  jax.ShapeDtypeStruct((2 * npad,), jnp.float32)),
        mesh=_sc_mesh(),
        scratch_types=[
            pltpu.VMEM((nch, _CH), jnp.int32),
            rows_t,
            pltpu.VMEM((_CH,), jnp.float32),
            pltpu.VMEM((per,), jnp.float32),
            sh_agg_t,
            pltpu.VMEM_SHARED((npad,), jnp.float32),
            pltpu.SemaphoreType.DMA,
            pltpu.SemaphoreType.DMA,
        ],
        compiler_params=_SC_PARAMS,
    )
    def k(msg_hbm, dst_hbm, zrow_hbm, zcnt_hbm, agg_hbm, cnt_hbm,
          idx_v, rows_v, ones_v, dummy_v, sh_agg, sh_cnt, sem_a, sem_c):
        cid = lax.axis_index("c")
        sid = lax.axis_index("s")
        wid = sid * 2 + cid
        base = wid * per

        for i in range(_CH // _LANES):
            ones_v[pl.ds(i * _LANES, _LANES)] = jnp.ones((_LANES,), jnp.float32)

        r0 = sid * rpt
        pltpu.sync_copy(dst_hbm.at[pl.ds(wid * nch, nch)], idx_v)
        msg_cp = pltpu.async_copy(msg_hbm.at[pl.ds(base, per)], rows_v, sem_a)
        if wide:
            pltpu.sync_copy(zrow_hbm.at[pl.ds(r0, rpt)], sh_agg.at[pl.ds(r0, rpt)])
        else:
            pltpu.sync_copy(zcnt_hbm.at[pl.ds(r0, rpt)], sh_agg.at[pl.ds(r0, rpt)])
        pltpu.sync_copy(zcnt_hbm.at[pl.ds(r0, rpt)], sh_cnt.at[pl.ds(r0, rpt)])
        plsc.subcore_barrier()
        msg_cp.wait()

        def body(j, carry):
            sl = pl.ds(j * _CH, _CH)
            pltpu.async_copy(rows_v.at[sl], sh_agg.at[idx_v.at[j]], sem_a,
                             add=True)
            pltpu.async_copy(ones_v, sh_cnt.at[idx_v.at[j]], sem_c, add=True)
            return carry

        lax.fori_loop(0, nch, body, 0)
        pltpu.make_async_copy(msg_hbm.at[pl.ds(base, per)], rows_v,
                              sem_a).wait()
        pltpu.make_async_copy(zcnt_hbm.at[pl.ds(0, per)], dummy_v,
                              sem_c).wait()
        plsc.subcore_barrier()

        o0 = cid * npad + r0
        pltpu.sync_copy(sh_agg.at[pl.ds(r0, rpt)], agg_hbm.at[pl.ds(o0, rpt)])
        pltpu.sync_copy(sh_cnt.at[pl.ds(r0, rpt)], cnt_hbm.at[pl.ds(o0, rpt)])

    zrow = jnp.zeros((npad, _HID), jnp.float32)
    zcnt = jnp.zeros((npad,), jnp.float32)
    return k(msg, dst, zrow, zcnt)


def _tc_msg(stim2d, xs, w1, b1, w2, b2, out_f):
    ep, d = xs.shape[0], w2.shape[0]
    wx = xs.shape[1]
    blk = 2048
    grid = ep // blk

    def body(stim_ref, xs_ref, w1_ref, b1_ref, w2_ref, b2_ref, out_ref):
        s = stim_ref[...]
        h = jnp.maximum(s * w1_ref[...] + b1_ref[...], 0.0)
        w = jnp.dot(h.astype(jnp.bfloat16), w2_ref[...].astype(jnp.bfloat16),
                    preferred_element_type=jnp.float32) + b2_ref[...]
        jj = lax.broadcasted_iota(jnp.int32, (wx, d), 1)
        ii = lax.broadcasted_iota(jnp.int32, (wx, d), 0)
        r = (jj // out_f == ii).astype(jnp.float32)
        xr = jnp.dot(xs_ref[...], r, preferred_element_type=jnp.float32)
        jo = lax.broadcasted_iota(jnp.int32, (d, out_f), 0)
        oo = lax.broadcasted_iota(jnp.int32, (d, out_f), 1)
        sred = (jo % out_f == oo).astype(jnp.float32)
        out_ref[...] = jnp.dot(xr * w, sred,
                               preferred_element_type=jnp.float32)

    return pl.pallas_call(
        body,
        grid=(grid,),
        in_specs=[
            pl.BlockSpec((blk, 1), lambda i: (i, 0)),
            pl.BlockSpec((blk, wx), lambda i: (i, 0)),
            pl.BlockSpec((1, d), lambda i: (0, 0)),
            pl.BlockSpec((1, d), lambda i: (0, 0)),
            pl.BlockSpec((d, d), lambda i: (0, 0)),
            pl.BlockSpec((1, d), lambda i: (0, 0)),
        ],
        out_specs=pl.BlockSpec((blk, out_f), lambda i: (i, 0)),
        out_shape=jax.ShapeDtypeStruct((ep, out_f), jnp.float32),
    )(stim2d, xs, w1, b1.reshape(1, d), w2, b2.reshape(1, d))


def _tc_norm(a0, a1, c0, c1, bias, g, beta):
    npad = a0.shape[0]
    blk = 512

    def body(a0_ref, a1_ref, c0_ref, c1_ref, bi_ref, g_ref, be_ref, out_ref):
        agg = a0_ref[...] + a1_ref[...]
        cnt = jnp.maximum(c0_ref[...] + c1_ref[...], 1.0)
        h = agg / cnt + bi_ref[...]
        mu = jnp.mean(h, axis=-1, keepdims=True)
        var = jnp.mean((h - mu) ** 2, axis=-1, keepdims=True)
        hn = (h - mu) / jnp.sqrt(var + 1e-5) * g_ref[...] + be_ref[...]
        out_ref[...] = jnp.maximum(hn, 0.0)

    return pl.pallas_call(
        body,
        grid=(npad // blk,),
        in_specs=[
            pl.BlockSpec((blk, _HID), lambda i: (i, 0)),
            pl.BlockSpec((blk, _HID), lambda i: (i, 0)),
            pl.BlockSpec((blk, 1), lambda i: (i, 0)),
            pl.BlockSpec((blk, 1), lambda i: (i, 0)),
            pl.BlockSpec((1, _HID), lambda i: (0, 0)),
            pl.BlockSpec((1, _HID), lambda i: (0, 0)),
            pl.BlockSpec((1, _HID), lambda i: (0, 0)),
        ],
        out_specs=pl.BlockSpec((blk, _HID), lambda i: (i, 0)),
        out_shape=jax.ShapeDtypeStruct((npad, _HID), jnp.float32),
    )(a0, a1, c0, c1, bias.reshape(1, _HID), g.reshape(1, _HID),
      beta.reshape(1, _HID))


def _tc_final(a0, a1, c0, c1, bias):
    npad = a0.shape[0]
    blk = 512

    def body(a0_ref, a1_ref, c0_ref, c1_ref, bi_ref, out_ref):
        agg = a0_ref[...] + a1_ref[...]
        cnt = jnp.maximum(c0_ref[...] + c1_ref[...], 1.0)
        h = agg / cnt + bi_ref[...]
        out_ref[...] = jnp.log1p(jnp.exp(-jnp.abs(h))) + jnp.maximum(h, 0.0)

    return pl.pallas_call(
        body,
        grid=(npad // blk,),
        in_specs=[
            pl.BlockSpec((blk, 1), lambda i: (i, 0)),
            pl.BlockSpec((blk, 1), lambda i: (i, 0)),
            pl.BlockSpec((blk, 1), lambda i: (i, 0)),
            pl.BlockSpec((blk, 1), lambda i: (i, 0)),
            pl.BlockSpec((1, 1), lambda i: (0, 0)),
        ],
        out_specs=pl.BlockSpec((blk, 1), lambda i: (i, 0)),
        out_shape=jax.ShapeDtypeStruct((npad, 1), jnp.float32),
    )(a0, a1, c0, c1, bias.reshape(1, 1))


def kernel(x, edge_index1, stim1, edge_index2, stim2, edge_index3, stim3,
           mlp1_W1, mlp1_b1, mlp1_W2, mlp1_b2, conv1_b, norm1_g, norm1_b,
           mlp2_W1, mlp2_b1, mlp2_W2, mlp2_b2, conv2_b, norm2_g, norm2_b,
           mlp3_W1, mlp3_b1, mlp3_W2, mlp3_b2, conv3_b):
    n, in_f = x.shape
    e = edge_index1.shape[1]
    npad = _ceil_to(n + 1, 512)
    epad = _ceil_to(e, _NW * _CH)

    def pad_edges(ei, stim):
        pe = epad - e
        src = jnp.concatenate([ei[0], jnp.zeros((pe,), jnp.int32)])
        dst = jnp.concatenate([ei[1], jnp.full((pe,), n, jnp.int32)])
        st = jnp.concatenate([stim, jnp.zeros((pe,), jnp.float32)])
        return (src.reshape(epad // _CH, _CH), dst.reshape(epad // _CH, _CH),
                st.reshape(epad, 1))

    src1, dst1, st1 = pad_edges(edge_index1, stim1)
    src2, dst2, st2 = pad_edges(edge_index2, stim2)
    src3, dst3, st3 = pad_edges(edge_index3, stim3)

    xpad = jnp.zeros((npad, _HID), jnp.float32).at[:n, :in_f].set(x)

    def layer(table, src, dst, st, w1, b1, w2, b2):
        xs = _sc_gather(table, src)
        msg = _tc_msg(st, xs, w1, b1, w2, b2, _HID)
        aggp, cntp = _sc_scatter(msg, dst, npad, wide=True)
        return aggp, cntp

    def split(aggp, cntp, wide):
        if wide:
            a0, a1 = aggp[:npad], aggp[npad:]
        else:
            a0, a1 = aggp[:npad, None], aggp[npad:, None]
        c0, c1 = cntp[:npad, None], cntp[npad:, None]
        return a0, a1, c0, c1

    aggp, cntp = layer(xpad, src1, dst1, st1, mlp1_W1, mlp1_b1, mlp1_W2, mlp1_b2)
    h1 = _tc_norm(*split(aggp, cntp, True), conv1_b, norm1_g, norm1_b)

    aggp, cntp = layer(h1, src2, dst2, st2, mlp2_W1, mlp2_b1, mlp2_W2, mlp2_b2)
    h2 = _tc_norm(*split(aggp, cntp, True), conv2_b, norm2_g, norm2_b)

    xs3 = _sc_gather(h2, src3)
    msg3 = _tc_msg(st3, xs3, mlp3_W1, mlp3_b1, mlp3_W2, mlp3_b2, 1)
    aggp3, cntp3 = _sc_scatter(msg3.reshape(epad), dst3, npad, wide=False)
    out = _tc_final(*split(aggp3, cntp3, False), conv3_b)
    return out[:n]
